# baseline (speedup 1.0000x reference)
import functools

import jax
import jax.numpy as jnp
from jax import lax
from jax.experimental import pallas as pl
from jax.experimental.pallas import tpu as pltpu
from jax.experimental.pallas import tpu_sc as plsc

_B, _S, _D = 4096, 200, 64
_SCALE = float(_D) ** 0.5
_NC, _NS, _L = 2, 16, 16
_NW = _NC * _NS
_ROWS = _B * _S
_RPW = _ROWS // _NW
_C = 512
_NCHUNK = _RPW // _C


def _emb_body(ids_hbm, table_hbm, out_hbm, idx_v, rows_v, sem):
    wid = lax.axis_index("s") * _NC + lax.axis_index("c")
    base = wid * _RPW

    def chunk(g, carry):
        off = base + g * _C
        pltpu.sync_copy(ids_hbm.at[pl.ds(off, _C)], idx_v)
        pltpu.async_copy(table_hbm.at[idx_v], rows_v, sem).wait()

        def scale_row(r, c2):
            for j in range(_D // _L):
                sl = pl.ds(j * _L, _L)
                rows_v[r, sl] = rows_v[r, sl] * _SCALE
            return c2

        lax.fori_loop(0, _C, scale_row, 0)
        pltpu.sync_copy(rows_v, out_hbm.at[pl.ds(off, _C)])
        return carry

    lax.fori_loop(0, _NCHUNK, chunk, 0)


def kernel(input_ids, weight):
    ids = input_ids.reshape(_ROWS)
    mesh = plsc.VectorSubcoreMesh(core_axis_name="c", subcore_axis_name="s")
    run = functools.partial(
        pl.kernel,
        mesh=mesh,
        compiler_params=pltpu.CompilerParams(use_tc_tiling_on_sc=False),
        out_type=jax.ShapeDtypeStruct((_ROWS, _D), jnp.float32),
        scratch_types=[
            pltpu.VMEM((_C,), jnp.int32),
            pltpu.VMEM((_C, _D), jnp.float32),
            pltpu.SemaphoreType.DMA,
        ],
    )(_emb_body)
    out = run(ids, weight)
    return out.reshape(_B, _S, _D)
